# baseline (device time: 36762 ns/iter reference)
import jax
import jax.numpy as jnp
from jax import lax
from jax.experimental import pallas as pl
from jax.experimental.pallas import tpu as pltpu

N_DEV = 32
LOG2_N = 5


def kernel(x, router_W, route_idx, expert_W):
    del router_W
    n_tok, d_model = x.shape
    n_exp_local, _, d_out = expert_W.shape

    def body(x_ref, idx_ref, ew_ref, out_ref, comm_ref, send_sems, recv_sems):
        my = lax.axis_index("i")

        barrier = pltpu.get_barrier_semaphore()
        for r in range(LOG2_N):
            pl.semaphore_signal(
                barrier, inc=1,
                device_id=(my ^ (1 << r),),
                device_id_type=pl.DeviceIdType.MESH,
            )
        pl.semaphore_wait(barrier, LOG2_N)

        xb = x_ref[...].astype(jnp.bfloat16)
        idx = idx_ref[...]
        e0 = my * n_exp_local
        acc = jnp.zeros((n_tok, d_out), jnp.float32)
        for e in range(n_exp_local):
            w = ew_ref[e].astype(jnp.bfloat16)
            h = jnp.dot(xb, w, preferred_element_type=jnp.float32)
            acc = acc + jnp.where(idx == e0 + e, h, 0.0)
        out_ref[...] = acc

        for r in range(LOG2_N):
            partner = my ^ (1 << r)
            rdma = pltpu.make_async_remote_copy(
                src_ref=out_ref,
                dst_ref=comm_ref.at[r],
                send_sem=send_sems.at[r],
                recv_sem=recv_sems.at[r],
                device_id=(partner,),
                device_id_type=pl.DeviceIdType.MESH,
            )
            rdma.start()
            rdma.wait()
            out_ref[...] += comm_ref[r]

    return pl.pallas_call(
        body,
        out_shape=jax.ShapeDtypeStruct((n_tok, d_out), jnp.float32),
        in_specs=[
            pl.BlockSpec(memory_space=pltpu.VMEM),
            pl.BlockSpec(memory_space=pltpu.VMEM),
            pl.BlockSpec(memory_space=pltpu.VMEM),
        ],
        out_specs=pl.BlockSpec(memory_space=pltpu.VMEM),
        scratch_shapes=[
            pltpu.VMEM((LOG2_N, n_tok, d_out), jnp.float32),
            pltpu.SemaphoreType.DMA((LOG2_N,)),
            pltpu.SemaphoreType.DMA((LOG2_N,)),
        ],
        compiler_params=pltpu.CompilerParams(collective_id=0),
    )(x, route_idx, expert_W)


# device time: 26857 ns/iter; 1.3688x vs baseline; 1.3688x over previous
import jax
import jax.numpy as jnp
from jax import lax
from jax.experimental import pallas as pl
from jax.experimental.pallas import tpu as pltpu

N_DEV = 32
LOG2_N = 5


def kernel(x, router_W, route_idx, expert_W):
    del router_W
    n_tok, d_model = x.shape
    n_exp_local, _, d_out = expert_W.shape

    def body(x_ref, idx_ref, ew_ref, out_ref, acc_ref, comm_ref,
             send_sems, recv_sems):
        my = lax.axis_index("i")

        barrier = pltpu.get_barrier_semaphore()
        for r in range(LOG2_N):
            pl.semaphore_signal(
                barrier, inc=1,
                device_id=(my ^ (1 << r),),
                device_id_type=pl.DeviceIdType.MESH,
            )
        pl.semaphore_wait(barrier, LOG2_N)

        xb = x_ref[...].astype(jnp.bfloat16)
        idx = idx_ref[...]
        e0 = my * n_exp_local
        acc = jnp.zeros((n_tok, d_out), jnp.float32)
        for e in range(n_exp_local):
            w = ew_ref[e].astype(jnp.bfloat16)
            h = jnp.dot(xb, w, preferred_element_type=jnp.float32)
            acc = acc + jnp.where(idx == e0 + e, h, 0.0)
        acc_ref[0] = acc.astype(jnp.bfloat16)

        rdmas = []
        for r in range(LOG2_N):
            p = r & 1
            partner = my ^ (1 << r)
            rdma = pltpu.make_async_remote_copy(
                src_ref=acc_ref.at[p],
                dst_ref=comm_ref.at[r],
                send_sem=send_sems.at[r],
                recv_sem=recv_sems.at[r],
                device_id=(partner,),
                device_id_type=pl.DeviceIdType.MESH,
            )
            rdma.start()
            if r > 0:
                rdmas[r - 1].wait_send()
            rdma.wait_recv()
            acc_ref[1 - p] = acc_ref[p] + comm_ref[r]
            rdmas.append(rdma)
        rdmas[-1].wait_send()
        out_ref[...] = acc_ref[LOG2_N & 1].astype(jnp.float32)

    return pl.pallas_call(
        body,
        out_shape=jax.ShapeDtypeStruct((n_tok, d_out), jnp.float32),
        in_specs=[
            pl.BlockSpec(memory_space=pltpu.VMEM),
            pl.BlockSpec(memory_space=pltpu.VMEM),
            pl.BlockSpec(memory_space=pltpu.VMEM),
        ],
        out_specs=pl.BlockSpec(memory_space=pltpu.VMEM),
        scratch_shapes=[
            pltpu.VMEM((2, n_tok, d_out), jnp.bfloat16),
            pltpu.VMEM((LOG2_N, n_tok, d_out), jnp.bfloat16),
            pltpu.SemaphoreType.DMA((LOG2_N,)),
            pltpu.SemaphoreType.DMA((LOG2_N,)),
        ],
        compiler_params=pltpu.CompilerParams(collective_id=0),
    )(x, route_idx, expert_W)
